# baseline (speedup 1.0000x reference)
import functools

import jax
import jax.numpy as jnp
from jax import lax
from jax.experimental import pallas as pl
from jax.experimental.pallas import tpu as pltpu
from jax.experimental.pallas import tpu_sc as plsc

B, N, H = 32, 64, 32
D, F = 5, 3
E_ROWS = 1537
E_PAD = 1552
NSPA = 512
NVAR = 5 * D
SPA_BASE = NVAR * E_PAD
TBL_ROWS = (NVAR + 1) * E_PAD
P = B * N * N
NPASS = D * F + 1
NC, NS = 2, 16
NW = NC * NS
CHUNK = 1024
NCHUNKS = P // CHUNK
NCH_PER_W = NCHUNKS // NW
JS = CHUNK // 128


def _table_body(e_ref, w_ref, spa_ref, o_ref):
    e = e_ref[...]
    q = E_PAD // 4
    for k in range(NVAR + 1):
        if k < NVAR:
            scale = 1.0 / (3.0 * (k // D + 1))
            v = jnp.dot(e, w_ref[k % D],
                        preferred_element_type=jnp.float32) * scale
        else:
            v = spa_ref[...]
        v4 = v.reshape(q, 4, H)
        for m in range(4):
            o_ref[pl.ds(k * q, q), pl.ds(m * H, H)] = v4[:, m, :]


IDX_CPB = 16


def _idx_body(ed_ref, sp_ref, o_ref):
    w = IDX_CPB * CHUNK
    ed = ed_ref[0]
    sp = sp_ref[0]
    sp_ = jnp.where(sp == 0, 1, sp)
    sp_ = jnp.where(sp_ > 1, sp_ - 1, sp_)
    s = jnp.clip(sp_, 0, D)
    base = (s - 1) * (D * E_PAD)
    edt = ed.T
    dvec = (lax.broadcasted_iota(jnp.int32, (D * F, w), 0) // F) * E_PAD
    idx_edge = edt + dvec + base
    idx_spa = sp + SPA_BASE
    out = jnp.concatenate([idx_edge, idx_spa], axis=0)
    parts = [out[:, cc * CHUNK:(cc + 1) * CHUNK].reshape(NPASS * JS, 128)
             for cc in range(IDX_CPB)]
    o_ref[...] = jnp.concatenate(parts, axis=0)


def _asm_body(core_ref, ab_ref, t_ref, o_ref):
    x = core_ref[...]
    parts = []
    for c in range(4):
        parts.append(x[:, c, :, 0:64])
        parts.append(x[:, c, :, 64:128])
    ct = jnp.concatenate(parts, axis=1)
    ab = ab_ref[0]
    t = t_ref[0]
    ii = lax.broadcasted_iota(jnp.int32, (N + 1, N + 1), 0)
    jj = lax.broadcasted_iota(jnp.int32, (N + 1, N + 1), 1)
    border = jnp.logical_or(ii == 0, jj == 0).astype(jnp.float32)
    padded = jnp.pad(ct, ((0, 0), (1, 0), (1, 0)))
    o_ref[0] = 2.0 * ab[None] + t[:, None, None] * border[None] + padded


@functools.cache
def _get_sc_gather():
    mesh = plsc.VectorSubcoreMesh(
        core_axis_name="c", subcore_axis_name="s",
        num_cores=NC, num_subcores=NS)

    @functools.partial(
        pl.kernel,
        out_type=jax.ShapeDtypeStruct((H, NCHUNKS, CHUNK // 128, 128),
                                      jnp.float32),
        mesh=mesh,
        scratch_types=[
            pltpu.VMEM((NPASS * CHUNK,), jnp.int32),
            pltpu.VMEM((CHUNK, H), jnp.float32),
            pltpu.VMEM((CHUNK, H), jnp.float32),
            pltpu.VMEM((H, CHUNK // 128, 128), jnp.float32),
            pltpu.SemaphoreType.DMA,
            pltpu.SemaphoreType.DMA,
        ],
        compiler_params=pltpu.CompilerParams(
            use_tc_tiling_on_sc=False, needs_layout_passes=False),
    )
    def _sc_gather(tbl_hbm, idx_hbm, out_hbm, idx_v, acc0_v, acc1_v,
                   ct_v, sem, sem_st):
        wid = lax.axis_index("s") * NC + lax.axis_index("c")
        accs = [acc0_v, acc1_v]

        def load_idx(c):
            g = wid * NCH_PER_W + c
            pltpu.sync_copy(
                idx_hbm.at[pl.ds(g * NPASS * CHUNK, NPASS * CHUNK)], idx_v)

        def fire_init(c):
            return [
                pltpu.async_copy(
                    tbl_hbm.at[idx_v.at[pl.ds(0, CHUNK)]],
                    accs[c % 2],
                    sem,
                )
            ]

        def fire_adds(c):
            return [
                pltpu.async_copy(
                    tbl_hbm.at[idx_v.at[pl.ds(t * CHUNK, CHUNK)]],
                    accs[c % 2],
                    sem,
                    add=True,
                )
                for t in range(1, NPASS)
            ]

        def transpose(c):
            acc_v = accs[c % 2]

            def tr_body(h, carry2):
                hv = jnp.full((16,), h, jnp.int32)
                for r in range(CHUNK // 128):
                    for k in range(8):
                        rows = lax.iota(jnp.int32, 16) + (
                            r * 64 + (k // 4) * 512 + (k % 4) * 16)
                        vec = plsc.load_gather(acc_v, [rows, hv])
                        ct_v[h, r, pl.ds(k * 16, 16)] = vec
                return carry2

            lax.fori_loop(0, H, tr_body, 0)

        def fire_store(c):
            g = wid * NCH_PER_W + c
            return pltpu.async_copy(ct_v, out_hbm.at[:, g], sem_st)

        load_idx(0)
        for cp in fire_init(0):
            cp.wait()
        adds = fire_adds(0)
        st = None
        for c in range(NCH_PER_W):
            for cp in adds:
                cp.wait()
            if c + 1 < NCH_PER_W:
                load_idx(c + 1)
                for cp in fire_init(c + 1):
                    cp.wait()
                adds = fire_adds(c + 1)
            if st is not None:
                st.wait()
            transpose(c)
            st = fire_store(c)
        st.wait()

    return _sc_gather


def kernel(input_nodes, attn_bias, spatial_pos, input_edges, attn_edge_type,
           edge_encoder_weight, edge_dis_encoder_weight,
           spatial_pos_encoder_weight, graph_token_virtual_distance_weight):
    del input_nodes, attn_edge_type

    e_pad = jnp.pad(edge_encoder_weight, ((0, E_PAD - E_ROWS), (0, 0)))
    spa_pad = jnp.pad(spatial_pos_encoder_weight, ((0, E_PAD - NSPA), (0, 0)))
    dis_w = edge_dis_encoder_weight.reshape(-1, H, H)[:D]
    scaled = pl.pallas_call(
        _table_body,
        grid=(1,),
        in_specs=[
            pl.BlockSpec((E_PAD, H), lambda k: (0, 0)),
            pl.BlockSpec((D, H, H), lambda k: (0, 0, 0)),
            pl.BlockSpec((E_PAD, H), lambda k: (0, 0)),
        ],
        out_specs=pl.BlockSpec((TBL_ROWS // 4, 128), lambda k: (0, 0)),
        out_shape=jax.ShapeDtypeStruct((TBL_ROWS // 4, 128), jnp.float32),
    )(e_pad, dis_w, spa_pad)
    table = scaled.reshape(TBL_ROWS, H)

    edges_r = input_edges.reshape(NCHUNKS // IDX_CPB, IDX_CPB * CHUNK,
                                  D * F).astype(jnp.int32)
    spat_r = spatial_pos.reshape(NCHUNKS // IDX_CPB, 1,
                                 IDX_CPB * CHUNK).astype(jnp.int32)
    idx = pl.pallas_call(
        _idx_body,
        grid=(NCHUNKS // IDX_CPB,),
        in_specs=[
            pl.BlockSpec((1, IDX_CPB * CHUNK, D * F), lambda k: (k, 0, 0)),
            pl.BlockSpec((1, 1, IDX_CPB * CHUNK), lambda k: (k, 0, 0)),
        ],
        out_specs=pl.BlockSpec((IDX_CPB * NPASS * JS, 128), lambda k: (k, 0)),
        out_shape=jax.ShapeDtypeStruct((NCHUNKS * NPASS * JS, 128),
                                       jnp.int32),
    )(edges_r, spat_r)

    core_t = _get_sc_gather()(table, idx.reshape(-1))

    out = pl.pallas_call(
        _asm_body,
        grid=(B,),
        in_specs=[
            pl.BlockSpec((H, NCH_PER_W, CHUNK // 128, 128),
                         lambda b: (0, b, 0, 0)),
            pl.BlockSpec((1, N + 1, N + 1), lambda b: (b, 0, 0)),
            pl.BlockSpec((1, H), lambda b: (0, 0)),
        ],
        out_specs=pl.BlockSpec((1, H, N + 1, N + 1), lambda b: (b, 0, 0, 0)),
        out_shape=jax.ShapeDtypeStruct((B, H, N + 1, N + 1), jnp.float32),
    )(core_t, attn_bias, graph_token_virtual_distance_weight)
    return out


# speedup vs baseline: 1.0138x; 1.0138x over previous
import functools

import jax
import jax.numpy as jnp
from jax import lax
from jax.experimental import pallas as pl
from jax.experimental.pallas import tpu as pltpu
from jax.experimental.pallas import tpu_sc as plsc

B, N, H = 32, 64, 32
D, F = 5, 3
E_ROWS = 1537
E_PAD = 1544
NSPA = 512
NVAR = 5 * D
SPA_BASE = NVAR * E_PAD
TBL_ROWS = (NVAR + 1) * E_PAD
P = B * N * N
NPASS = D * F + 1
NC, NS = 2, 16
NW = NC * NS
CHUNK = 1024
NCHUNKS = P // CHUNK
NCH_PER_W = NCHUNKS // NW
JS = CHUNK // 128


def _table_body(e_ref, w_ref, spa_ref, o_ref):
    e = e_ref[...]
    for k in range(NVAR):
        scale = 1.0 / (3.0 * (k // D + 1))
        o_ref[k] = jnp.dot(e, w_ref[k % D],
                           preferred_element_type=jnp.float32) * scale
    o_ref[NVAR] = spa_ref[...]


IDX_CPB = 16


def _idx_body(ed_ref, sp_ref, o_ref):
    w = IDX_CPB * CHUNK
    ed = ed_ref[0]
    sp = sp_ref[0]
    sp_ = jnp.where(sp == 0, 1, sp)
    sp_ = jnp.where(sp_ > 1, sp_ - 1, sp_)
    s = jnp.clip(sp_, 0, D)
    base = (s - 1) * (D * E_PAD)
    edt = ed.T
    dvec = (lax.broadcasted_iota(jnp.int32, (D * F, w), 0) // F) * E_PAD
    idx_edge = edt + dvec + base
    idx_spa = sp + SPA_BASE
    out = jnp.concatenate([idx_edge, idx_spa], axis=0)
    parts = [out[:, cc * CHUNK:(cc + 1) * CHUNK].reshape(NPASS * JS, 128)
             for cc in range(IDX_CPB)]
    o_ref[...] = jnp.concatenate(parts, axis=0)


def _asm_body(core_ref, ab_ref, t_ref, o_ref):
    x = core_ref[...]
    parts = []
    for c in range(4):
        parts.append(x[:, c * 8:(c + 1) * 8, 0:64])
        parts.append(x[:, c * 8:(c + 1) * 8, 64:128])
    ct = jnp.concatenate(parts, axis=1)
    ab = ab_ref[0]
    t = t_ref[0]
    ii = lax.broadcasted_iota(jnp.int32, (N + 1, N + 1), 0)
    jj = lax.broadcasted_iota(jnp.int32, (N + 1, N + 1), 1)
    border = jnp.logical_or(ii == 0, jj == 0).astype(jnp.float32)
    padded = jnp.pad(ct, ((0, 0), (1, 0), (1, 0)))
    o_ref[0] = 2.0 * ab[None] + t[:, None, None] * border[None] + padded


@functools.cache
def _get_sc_gather():
    mesh = plsc.VectorSubcoreMesh(
        core_axis_name="c", subcore_axis_name="s",
        num_cores=NC, num_subcores=NS)

    @functools.partial(
        pl.kernel,
        out_type=jax.ShapeDtypeStruct((H, P // 128, 128), jnp.float32),
        mesh=mesh,
        scratch_types=[
            pltpu.VMEM((NPASS * CHUNK,), jnp.int32),
            pltpu.VMEM((CHUNK, H), jnp.float32),
            pltpu.VMEM((CHUNK, H), jnp.float32),
            pltpu.VMEM((H, CHUNK // 128, 128), jnp.float32),
            pltpu.SemaphoreType.DMA,
            pltpu.SemaphoreType.DMA,
        ],
        compiler_params=pltpu.CompilerParams(
            use_tc_tiling_on_sc=False, needs_layout_passes=False),
    )
    def _sc_gather(tbl_hbm, idx_hbm, out_hbm, idx_v, acc0_v, acc1_v,
                   ct_v, sem, sem_st):
        wid = lax.axis_index("s") * NC + lax.axis_index("c")
        accs = [acc0_v, acc1_v]

        def load_idx(c):
            g = wid * NCH_PER_W + c
            pltpu.sync_copy(
                idx_hbm.at[pl.ds(g * NPASS * CHUNK, NPASS * CHUNK)], idx_v)

        def fire_init(c):
            return [
                pltpu.async_copy(
                    tbl_hbm.at[idx_v.at[pl.ds(0, CHUNK)]],
                    accs[c % 2],
                    sem,
                )
            ]

        def fire_adds(c):
            return [
                pltpu.async_copy(
                    tbl_hbm.at[idx_v.at[pl.ds(t * CHUNK, CHUNK)]],
                    accs[c % 2],
                    sem,
                    add=True,
                )
                for t in range(1, NPASS)
            ]

        def transpose(c):
            acc_v = accs[c % 2]

            def tr_body(h, carry2):
                hv = jnp.full((16,), h, jnp.int32)
                for r in range(CHUNK // 128):
                    for k in range(8):
                        rows = lax.iota(jnp.int32, 16) + (
                            r * 64 + (k // 4) * 512 + (k % 4) * 16)
                        vec = plsc.load_gather(acc_v, [rows, hv])
                        ct_v[h, r, pl.ds(k * 16, 16)] = vec
                return carry2

            lax.fori_loop(0, H, tr_body, 0)

        def fire_store(c):
            g = wid * NCH_PER_W + c
            return pltpu.async_copy(
                ct_v,
                out_hbm.at[:, pl.ds(g * (CHUNK // 128), CHUNK // 128)],
                sem_st)

        load_idx(0)
        for cp in fire_init(0):
            cp.wait()
        adds = fire_adds(0)
        st = None
        for c in range(NCH_PER_W):
            for cp in adds:
                cp.wait()
            if c + 1 < NCH_PER_W:
                load_idx(c + 1)
                for cp in fire_init(c + 1):
                    cp.wait()
                adds = fire_adds(c + 1)
            if st is not None:
                st.wait()
            transpose(c)
            st = fire_store(c)
        st.wait()

    return _sc_gather


def kernel(input_nodes, attn_bias, spatial_pos, input_edges, attn_edge_type,
           edge_encoder_weight, edge_dis_encoder_weight,
           spatial_pos_encoder_weight, graph_token_virtual_distance_weight):
    del input_nodes, attn_edge_type

    e_pad = jnp.pad(edge_encoder_weight, ((0, E_PAD - E_ROWS), (0, 0)))
    spa_pad = jnp.pad(spatial_pos_encoder_weight, ((0, E_PAD - NSPA), (0, 0)))
    dis_w = edge_dis_encoder_weight.reshape(-1, H, H)[:D]
    scaled = pl.pallas_call(
        _table_body,
        grid=(1,),
        in_specs=[
            pl.BlockSpec((E_PAD, H), lambda k: (0, 0)),
            pl.BlockSpec((D, H, H), lambda k: (0, 0, 0)),
            pl.BlockSpec((E_PAD, H), lambda k: (0, 0)),
        ],
        out_specs=pl.BlockSpec((NVAR + 1, E_PAD, H), lambda k: (0, 0, 0)),
        out_shape=jax.ShapeDtypeStruct((NVAR + 1, E_PAD, H), jnp.float32),
    )(e_pad, dis_w, spa_pad)
    table = scaled.reshape(TBL_ROWS, H)

    edges_r = input_edges.reshape(NCHUNKS // IDX_CPB, IDX_CPB * CHUNK,
                                  D * F).astype(jnp.int32)
    spat_r = spatial_pos.reshape(NCHUNKS // IDX_CPB, 1,
                                 IDX_CPB * CHUNK).astype(jnp.int32)
    idx = pl.pallas_call(
        _idx_body,
        grid=(NCHUNKS // IDX_CPB,),
        in_specs=[
            pl.BlockSpec((1, IDX_CPB * CHUNK, D * F), lambda k: (k, 0, 0)),
            pl.BlockSpec((1, 1, IDX_CPB * CHUNK), lambda k: (k, 0, 0)),
        ],
        out_specs=pl.BlockSpec((IDX_CPB * NPASS * JS, 128), lambda k: (k, 0)),
        out_shape=jax.ShapeDtypeStruct((NCHUNKS * NPASS * JS, 128),
                                       jnp.int32),
    )(edges_r, spat_r)

    core_t = _get_sc_gather()(table, idx.reshape(-1))

    out = pl.pallas_call(
        _asm_body,
        grid=(B,),
        in_specs=[
            pl.BlockSpec((H, N * N // 128, 128), lambda b: (0, b, 0)),
            pl.BlockSpec((1, N + 1, N + 1), lambda b: (b, 0, 0)),
            pl.BlockSpec((1, H), lambda b: (0, 0)),
        ],
        out_specs=pl.BlockSpec((1, H, N + 1, N + 1), lambda b: (b, 0, 0, 0)),
        out_shape=jax.ShapeDtypeStruct((B, H, N + 1, N + 1), jnp.float32),
    )(core_t, attn_bias, graph_token_virtual_distance_weight)
    return out
